# Initial kernel scaffold; baseline (speedup 1.0000x reference)
import jax
import jax.numpy as jnp
from jax.experimental import pallas as pl


def kernel(x, pos, edge_index, node_subnode_index, subgraph_edge_index, subnode_node_index, batch, params):
    raise NotImplementedError("write your pallas kernel here")


# speedup vs baseline: 1.5794x; 1.5794x over previous
import functools

import jax
import jax.numpy as jnp
from jax import lax
from jax.experimental import pallas as pl
from jax.experimental.pallas import tpu as pltpu
from jax.experimental.pallas import tpu_sc as plsc

N = 10000
E = 160000
H = 128
DEPTH = 2
NGRAPH = 64

NW = 32
N_PAD = 10240
E_PAD = 163840
EPW = E_PAD // NW
CHUNK = 128
POS_IDX = 8 * E_PAD
PPW = POS_IDX // NW
POOL_ACC = 128

def _dot(a, b):
    return jnp.dot(a, b, preferred_element_type=jnp.float32)


def _ln_swish(x, g, b):
    mu = jnp.mean(x, axis=-1, keepdims=True)
    xc = x - mu
    v = jnp.mean(xc * xc, axis=-1, keepdims=True)
    y = xc / jnp.sqrt(v + 1e-5) * g + b
    return y * jax.nn.sigmoid(y)


_MESH = plsc.VectorSubcoreMesh(core_axis_name="c", subcore_axis_name="s")
_SC_PARAMS = pltpu.CompilerParams(needs_layout_passes=False)


def _gather_pair_body(p_hbm, q_hbm, di_hbm, si_hbm, s_hbm, t_hbm,
                      di_c, si_c, rowp, rowq, sem):
    c = lax.axis_index("c")
    s = lax.axis_index("s")
    base = (c * 16 + s) * EPW

    def step(i, carry):
        off = base + i * CHUNK
        pltpu.sync_copy(di_hbm.at[pl.ds(off, CHUNK)], di_c)
        pltpu.sync_copy(si_hbm.at[pl.ds(off, CHUNK)], si_c)
        cp1 = pltpu.async_copy(p_hbm.at[di_c], rowp, sem)
        cp1.wait()
        cp2 = pltpu.async_copy(q_hbm.at[si_c], rowq, sem)
        cp2.wait()
        pltpu.sync_copy(rowp, s_hbm.at[pl.ds(off, CHUNK)])
        pltpu.sync_copy(rowq, t_hbm.at[pl.ds(off, CHUNK)])
        return carry

    lax.fori_loop(0, EPW // CHUNK, step, 0)


_sc_gather_pair = functools.partial(
    pl.kernel,
    out_type=[jax.ShapeDtypeStruct((E_PAD, H), jnp.float32),
              jax.ShapeDtypeStruct((E_PAD, H), jnp.float32)],
    mesh=_MESH,
    compiler_params=_SC_PARAMS,
    scratch_types=[
        pltpu.VMEM((CHUNK,), jnp.int32),
        pltpu.VMEM((CHUNK,), jnp.int32),
        pltpu.VMEM((CHUNK, H), jnp.float32),
        pltpu.VMEM((CHUNK, H), jnp.float32),
        pltpu.SemaphoreType.DMA,
    ],
)(_gather_pair_body)


D2_TOT = 4 * E_PAD
D2PW = D2_TOT // NW


def _d2_body(px_hbm, py_hbm, pz_hbm, di_hbm, si_hbm, out_hbm,
             px, py, pz, di_c, si_c, d2buf, sem):
    c = lax.axis_index("c")
    s = lax.axis_index("s")
    base = (c * 16 + s) * D2PW
    pltpu.sync_copy(px_hbm, px)
    pltpu.sync_copy(py_hbm, py)
    pltpu.sync_copy(pz_hbm, pz)

    def step(i, carry):
        off = base + i * CHUNK
        pltpu.sync_copy(di_hbm.at[pl.ds(off, CHUNK)], di_c)
        pltpu.sync_copy(si_hbm.at[pl.ds(off, CHUNK)], si_c)
        for j in range(CHUNK // 16):
            vd = di_c[pl.ds(j * 16, 16)]
            vs = si_c[pl.ds(j * 16, 16)]
            dx = plsc.load_gather(px, [vd]) - plsc.load_gather(px, [vs])
            dy = plsc.load_gather(py, [vd]) - plsc.load_gather(py, [vs])
            dz = plsc.load_gather(pz, [vd]) - plsc.load_gather(pz, [vs])
            d2buf[pl.ds(j * 16, 16)] = dx * dx + dy * dy + dz * dz
        pltpu.sync_copy(d2buf, out_hbm.at[pl.ds(off, CHUNK)])
        return carry

    lax.fori_loop(0, D2PW // CHUNK, step, 0)


_sc_d2 = functools.partial(
    pl.kernel,
    out_type=jax.ShapeDtypeStruct((D2_TOT,), jnp.float32),
    mesh=_MESH,
    compiler_params=_SC_PARAMS,
    scratch_types=[
        pltpu.VMEM((N_PAD,), jnp.float32),
        pltpu.VMEM((N_PAD,), jnp.float32),
        pltpu.VMEM((N_PAD,), jnp.float32),
        pltpu.VMEM((CHUNK,), jnp.int32),
        pltpu.VMEM((CHUNK,), jnp.int32),
        pltpu.VMEM((CHUNK,), jnp.float32),
        pltpu.SemaphoreType.DMA,
    ],
)(_d2_body)


def _make_scatter(n_in, n_acc, chunk):
    per_w = n_in // NW
    steps = per_w // chunk
    rpt = n_acc // 16

    def body(val_hbm, idx_hbm, zeros_hbm, out_hbm, idx_c, vbuf, acc, sem):
        c = lax.axis_index("c")
        s = lax.axis_index("s")
        base = (c * 16 + s) * per_w
        pltpu.sync_copy(zeros_hbm.at[pl.ds(s * rpt, rpt)],
                        acc.at[pl.ds(s * rpt, rpt)])
        plsc.subcore_barrier()

        def step(i, carry):
            off = base + i * chunk
            pltpu.sync_copy(val_hbm.at[pl.ds(off, chunk)], vbuf)
            pltpu.sync_copy(idx_hbm.at[pl.ds(off, chunk)], idx_c)
            pltpu.sync_copy(vbuf, acc.at[idx_c], add=True)
            return carry

        lax.fori_loop(0, steps, step, 0)
        plsc.subcore_barrier()
        pltpu.sync_copy(acc.at[pl.ds(s * rpt, rpt)],
                        out_hbm.at[c, pl.ds(s * rpt, rpt)])

    return functools.partial(
        pl.kernel,
        out_type=jax.ShapeDtypeStruct((2, n_acc, H), jnp.float32),
        mesh=_MESH,
        compiler_params=_SC_PARAMS,
        scratch_types=[
            pltpu.VMEM((chunk,), jnp.int32),
            pltpu.VMEM((chunk, H), jnp.float32),
            pltpu.VMEM_SHARED((n_acc, H), jnp.float32),
            pltpu.SemaphoreType.DMA,
        ],
    )(body)


_sc_scatter_node = _make_scatter(E_PAD, N_PAD, CHUNK)
_sc_scatter_pool = _make_scatter(N_PAD, POOL_ACC, 64)


BLK_E = 512
BLK_N = 512


def _vec_spec(grid_fn=None):
    if grid_fn is None:
        grid_fn = lambda i: (0, 0)
    return pl.BlockSpec((1, H), grid_fn)


def _wspec():
    return pl.BlockSpec((H, H), lambda i: (0, 0))


def _emb_kernel(x_ref, ew_ref, eb_ref, w1a_ref, w1b_ref,
                h_ref, p_ref, q_ref):
    h = _dot(x_ref[...], ew_ref[...]) + eb_ref[...]
    h_ref[...] = h
    p_ref[...] = _dot(h, w1a_ref[...])
    q_ref[...] = _dot(h, w1b_ref[...])


def _tc_emb(x_p, ew, eb, w1a, w1b):
    grid = (N_PAD // BLK_N,)
    row = pl.BlockSpec((BLK_N, H), lambda i: (i, 0))
    return pl.pallas_call(
        _emb_kernel,
        grid=grid,
        in_specs=[row, _wspec(), _vec_spec(), _wspec(), _wspec()],
        out_specs=[row, row, row],
        out_shape=[jax.ShapeDtypeStruct((N_PAD, H), jnp.float32)] * 3,
    )(x_p, ew, eb, w1a, w1b)


def _edge_kernel(s_ref, t_ref, d_ref, w1c_ref, b1_ref, g1_ref, be1_ref,
                 w2_ref, b2_ref, g2_ref, be2_ref, m_ref):
    d = jnp.sqrt(d_ref[...])
    m1 = s_ref[...] + t_ref[...] + d * w1c_ref[...] + b1_ref[...]
    m1 = _ln_swish(m1, g1_ref[...], be1_ref[...])
    m2 = _dot(m1, w2_ref[...]) + b2_ref[...]
    m_ref[...] = _ln_swish(m2, g2_ref[...], be2_ref[...])


def _tc_edge(S, T, d, w1c, b1, g1, be1, w2, b2, g2, be2):
    grid = (E_PAD // BLK_E,)
    row = pl.BlockSpec((BLK_E, H), lambda i: (i, 0))
    dspec = pl.BlockSpec((BLK_E, 1), lambda i: (i, 0))
    return pl.pallas_call(
        _edge_kernel,
        grid=grid,
        in_specs=[row, row, dspec, _vec_spec(), _vec_spec(), _vec_spec(),
                  _vec_spec(), _wspec(), _vec_spec(), _vec_spec(),
                  _vec_spec()],
        out_specs=row,
        out_shape=jax.ShapeDtypeStruct((E_PAD, H), jnp.float32),
    )(S, T, d, w1c, b1, g1, be1, w2, b2, g2, be2)


def _node_kernel(has_res, has_pq, *refs):
    i = iter(refs)
    h_ref = next(i)
    a0_ref = next(i)
    a1_ref = next(i)
    h0_ref = next(i) if has_res else None
    wu1a = next(i)[...]
    wu1b = next(i)[...]
    bu1 = next(i)[...]
    gu1 = next(i)[...]
    beu1 = next(i)[...]
    wu2 = next(i)[...]
    bu2 = next(i)[...]
    gu2 = next(i)[...]
    beu2 = next(i)[...]
    if has_pq:
        w1a = next(i)[...]
        w1b = next(i)[...]
    hn_ref = next(i)
    if has_pq:
        p_ref = next(i)
        q_ref = next(i)
    h = h_ref[...]
    agg = a0_ref[0] + a1_ref[0]
    u1 = _ln_swish(_dot(h, wu1a) + _dot(agg, wu1b) + bu1, gu1, beu1)
    u2 = _ln_swish(_dot(u1, wu2) + bu2, gu2, beu2)
    if has_res:
        u2 = u2 + h0_ref[...]
    hn_ref[...] = u2
    if has_pq:
        p_ref[...] = _dot(u2, w1a)
        q_ref[...] = _dot(u2, w1b)


def _tc_node(h, agg2, p, h0, nxt):
    has_res = h0 is not None
    has_pq = nxt is not None
    grid = (N_PAD // BLK_N,)
    row = pl.BlockSpec((BLK_N, H), lambda i: (i, 0))
    arow = pl.BlockSpec((1, BLK_N, H), lambda i: (0, i, 0))
    brow = pl.BlockSpec((1, BLK_N, H), lambda i: (1, i, 0))
    in_specs = [row, arow, brow]
    args = [h, agg2, agg2]
    if has_res:
        in_specs.append(row)
        args.append(h0)
    wu1a = p["upd_W1"][:H]
    wu1b = p["upd_W1"][H:]
    for spec, arg in [
        (_wspec(), wu1a), (_wspec(), wu1b),
        (_vec_spec(), p["upd_b1"].reshape(1, H)),
        (_vec_spec(), p["upd_g1"].reshape(1, H)),
        (_vec_spec(), p["upd_be1"].reshape(1, H)),
        (_wspec(), p["upd_W2"]),
        (_vec_spec(), p["upd_b2"].reshape(1, H)),
        (_vec_spec(), p["upd_g2"].reshape(1, H)),
        (_vec_spec(), p["upd_be2"].reshape(1, H)),
    ]:
        in_specs.append(spec)
        args.append(arg)
    if has_pq:
        for spec, arg in [
            (_wspec(), nxt["msg_W1"][:H]),
            (_wspec(), nxt["msg_W1"][H:2 * H]),
        ]:
            in_specs.append(spec)
            args.append(arg)
    n_out = 3 if has_pq else 1
    out = pl.pallas_call(
        functools.partial(_node_kernel, has_res, has_pq),
        grid=grid,
        in_specs=in_specs,
        out_specs=[row] * n_out,
        out_shape=[jax.ShapeDtypeStruct((N_PAD, H), jnp.float32)] * n_out,
    )(*args)
    if has_pq:
        return out[0], out[1], out[2]
    return out[0], None, None


def _pred_kernel(p2_ref, w1_ref, b1_ref, w2_ref, b2_ref, o_ref):
    pooled = p2_ref[0, :NGRAPH] + p2_ref[1, :NGRAPH]
    hid = jnp.maximum(_dot(pooled, w1_ref[...]) + b1_ref[...], 0.0)
    o_ref[...] = _dot(hid, w2_ref[...]) + b2_ref[...]


def _tc_pred(pooled2, w1, b1, w2, b2):
    return pl.pallas_call(
        _pred_kernel,
        in_specs=[
            pl.BlockSpec((2, POOL_ACC, H), lambda: (0, 0, 0)),
            _wspec_g(), pl.BlockSpec((1, H), lambda: (0, 0)),
            pl.BlockSpec((H, 1), lambda: (0, 0)),
            pl.BlockSpec((1, 1), lambda: (0, 0)),
        ],
        out_specs=pl.BlockSpec((NGRAPH, 1), lambda: (0, 0)),
        out_shape=jax.ShapeDtypeStruct((NGRAPH, 1), jnp.float32),
    )(pooled2, w1, b1.reshape(1, H), w2, b2.reshape(1, 1))


def _wspec_g():
    return pl.BlockSpec((H, H), lambda: (0, 0))


def kernel(x, pos, edge_index, node_subnode_index, subgraph_edge_index,
           subnode_node_index, batch, params):
    f32 = jnp.float32
    x_p = jnp.pad(x.astype(f32), ((0, N_PAD - N), (0, 0)))
    pos16 = jnp.pad(pos.astype(f32), ((0, N_PAD - N), (0, 13)))

    edge_sets = [edge_index, node_subnode_index, subgraph_edge_index,
                 subnode_node_index]
    dsts, srcs = [], []
    for es in edge_sets:
        es = es.astype(jnp.int32)
        pad = jnp.full((E_PAD - E,), N, jnp.int32)
        srcs.append(jnp.concatenate([es[0], pad]))
        dsts.append(jnp.concatenate([es[1], pad]))

    batch_pad = jnp.concatenate([
        batch.astype(jnp.int32),
        jnp.full((N_PAD - N,), NGRAPH, jnp.int32)])

    zeros_node = jnp.zeros((N_PAD, H), f32)
    zeros_pool = jnp.zeros((POOL_ACC, H), f32)

    pos_p = pos16[:, :3]
    dst_all = jnp.concatenate(dsts)
    src_all = jnp.concatenate(srcs)
    d2_flat = _sc_d2(pos_p[:, 0], pos_p[:, 1], pos_p[:, 2],
                     dst_all, src_all)
    d2_all = d2_flat.reshape(4, E_PAD, 1)
    d_list = [d2_all[t] for t in range(4)]

    prm = params
    lay = prm["layers"]

    first = lay[0][0]
    h, P, Q = _tc_emb(
        x_p, prm["emb_W"], prm["emb_b"].reshape(1, H),
        first["msg_W1"][:H], first["msg_W1"][H:2 * H])

    for l in range(DEPTH):
        h0 = h
        for t in range(4):
            p = lay[l][t]
            S, T = _sc_gather_pair(P, Q, dsts[t], srcs[t])
            m = _tc_edge(
                S, T, d_list[t],
                p["msg_W1"][2 * H:2 * H + 1],
                p["msg_b1"].reshape(1, H),
                p["msg_g1"].reshape(1, H), p["msg_be1"].reshape(1, H),
                p["msg_W2"], p["msg_b2"].reshape(1, H),
                p["msg_g2"].reshape(1, H), p["msg_be2"].reshape(1, H))
            agg2 = _sc_scatter_node(m, dsts[t], zeros_node)
            if t < 3:
                nxt = lay[l][t + 1]
            elif l + 1 < DEPTH:
                nxt = lay[l + 1][0]
            else:
                nxt = None
            h, P, Q = _tc_node(h, agg2, p, h0 if t == 3 else None, nxt)

    pooled2 = _sc_scatter_pool(h, batch_pad, zeros_pool)
    return _tc_pred(pooled2, prm["pred_W1"], prm["pred_b1"],
                    prm["pred_W2"], prm["pred_b2"])


# speedup vs baseline: 1.8657x; 1.1813x over previous
import functools

import jax
import jax.numpy as jnp
from jax import lax
from jax.experimental import pallas as pl
from jax.experimental.pallas import tpu as pltpu
from jax.experimental.pallas import tpu_sc as plsc

N = 10000
E = 160000
H = 128
DEPTH = 2
NGRAPH = 64

NW = 32
N_PAD = 10240
E_PAD = 163840
EPW = E_PAD // NW
CHUNK = 128
POS_IDX = 8 * E_PAD
PPW = POS_IDX // NW
POOL_ACC = 128

def _dot(a, b):
    return jnp.dot(a, b, preferred_element_type=jnp.float32)


def _ln_swish(x, g, b):
    mu = jnp.mean(x, axis=-1, keepdims=True)
    xc = x - mu
    v = jnp.mean(xc * xc, axis=-1, keepdims=True)
    y = xc / jnp.sqrt(v + 1e-5) * g + b
    return y * jax.nn.sigmoid(y)


_MESH = plsc.VectorSubcoreMesh(core_axis_name="c", subcore_axis_name="s")
_SC_PARAMS = pltpu.CompilerParams(needs_layout_passes=False)


def _gather_pair_body(p_hbm, q_hbm, di_hbm, si_hbm, s_hbm, t_hbm,
                      di0, si0, di1, si1, di2, si2, di3, si3,
                      rp0, rq0, rp1, rq1,
                      sd0, sd1, sd2, sd3, sg0, sg1, sw0, sw1):
    c = lax.axis_index("c")
    s = lax.axis_index("s")
    base = (c * 16 + s) * EPW
    steps = EPW // CHUNK
    IDX = [(di0, si0, sd0), (di1, si1, sd1), (di2, si2, sd2), (di3, si3, sd3)]
    ROW = [(rp0, rq0, sg0), (rp1, rq1, sg1)]
    SW = [sw0, sw1]

    def off(i):
        return base + i * CHUNK

    def idx_descs(isl, i):
        dib, sib, sem = IDX[isl]
        return (pltpu.make_async_copy(di_hbm.at[pl.ds(off(i), CHUNK)], dib, sem),
                pltpu.make_async_copy(si_hbm.at[pl.ds(off(i), CHUNK)], sib, sem))

    def gat_descs(rsl, isl):
        rp, rq, sem = ROW[rsl]
        dib, sib, _ = IDX[isl]
        return (pltpu.make_async_copy(p_hbm.at[dib], rp, sem),
                pltpu.make_async_copy(q_hbm.at[sib], rq, sem))

    def wb_descs(rsl, i):
        rp, rq, _ = ROW[rsl]
        return (pltpu.make_async_copy(rp, s_hbm.at[pl.ds(off(i), CHUNK)], SW[rsl]),
                pltpu.make_async_copy(rq, t_hbm.at[pl.ds(off(i), CHUNK)], SW[rsl]))

    def start(ds):
        for d in ds:
            d.start()

    def wait(ds):
        for d in ds:
            d.wait()

    start(idx_descs(0, 0))
    start(idx_descs(1, 1))

    def kbody(k, carry):
        for sl in range(4):
            i = 4 * k + sl
            rsl = sl % 2
            wait(idx_descs(sl, i))

            def wait_wb():
                wait(wb_descs(rsl, i - 2))
            if sl < 2:
                pl.when(k >= 1)(wait_wb)
            else:
                wait_wb()
            start(gat_descs(rsl, sl))

            prsl = (sl - 1) % 2
            pi = i - 1

            def fin_prev():
                wait(gat_descs(prsl, (sl - 1) % 4))
                start(wb_descs(prsl, pi))
            if sl == 0:
                pl.when(k >= 1)(fin_prev)
            else:
                fin_prev()

            nsl = (sl + 2) % 4

            def prefetch():
                start(idx_descs(nsl, i + 2))
            if sl < 2:
                prefetch()
            else:
                pl.when(k < steps // 4 - 1)(prefetch)
        return carry

    lax.fori_loop(0, steps // 4, kbody, 0)
    wait(gat_descs(1, 3))
    start(wb_descs(1, steps - 1))
    wait(wb_descs(0, steps - 2))
    wait(wb_descs(1, steps - 1))


_sc_gather_pair = functools.partial(
    pl.kernel,
    out_type=[jax.ShapeDtypeStruct((E_PAD, H), jnp.float32),
              jax.ShapeDtypeStruct((E_PAD, H), jnp.float32)],
    mesh=_MESH,
    compiler_params=_SC_PARAMS,
    scratch_types=(
        [pltpu.VMEM((CHUNK,), jnp.int32)] * 8
        + [pltpu.VMEM((CHUNK, H), jnp.float32)] * 4
        + [pltpu.SemaphoreType.DMA] * 8
    ),
)(_gather_pair_body)


D2_TOT = 4 * E_PAD
D2PW = D2_TOT // NW


def _d2_body(px_hbm, py_hbm, pz_hbm, di_hbm, si_hbm, out_hbm,
             px, py, pz, di0, si0, di1, si1, db0, db1,
             sf0, sf1, sw0, sw1):
    c = lax.axis_index("c")
    s = lax.axis_index("s")
    base = (c * 16 + s) * D2PW
    steps = D2PW // CHUNK
    pltpu.sync_copy(px_hbm, px)
    pltpu.sync_copy(py_hbm, py)
    pltpu.sync_copy(pz_hbm, pz)
    SL = [(di0, si0, db0, sf0, sw0), (di1, si1, db1, sf1, sw1)]

    def off(i):
        return base + i * CHUNK

    def fetch_descs(sl, i):
        di, si, _, sf, _ = SL[sl]
        return (pltpu.make_async_copy(di_hbm.at[pl.ds(off(i), CHUNK)], di, sf),
                pltpu.make_async_copy(si_hbm.at[pl.ds(off(i), CHUNK)], si, sf))

    def wb_desc(sl, i):
        db = SL[sl][2]
        return pltpu.make_async_copy(db, out_hbm.at[pl.ds(off(i), CHUNK)],
                                     SL[sl][4])

    def start(ds):
        for d in ds:
            d.start()

    def wait(ds):
        for d in ds:
            d.wait()

    start(fetch_descs(0, 0))

    def kbody(k, carry):
        for sl in range(2):
            i = 2 * k + sl
            if sl == 0:
                start(fetch_descs(1, i + 1))
            else:
                def pf():
                    start(fetch_descs(0, i + 1))
                pl.when(k < steps // 2 - 1)(pf)
            wait(fetch_descs(sl, i))

            def drain_wb():
                wb_desc(sl, i - 2).wait()
            pl.when(k >= 1)(drain_wb)
            di, si, db = SL[sl][0], SL[sl][1], SL[sl][2]
            for j in range(CHUNK // 16):
                vd = di[pl.ds(j * 16, 16)]
                vs = si[pl.ds(j * 16, 16)]
                dx = plsc.load_gather(px, [vd]) - plsc.load_gather(px, [vs])
                dy = plsc.load_gather(py, [vd]) - plsc.load_gather(py, [vs])
                dz = plsc.load_gather(pz, [vd]) - plsc.load_gather(pz, [vs])
                db[pl.ds(j * 16, 16)] = dx * dx + dy * dy + dz * dz
            wb_desc(sl, i).start()
        return carry

    lax.fori_loop(0, steps // 2, kbody, 0)
    wb_desc(0, steps - 2).wait()
    wb_desc(1, steps - 1).wait()


_sc_d2 = functools.partial(
    pl.kernel,
    out_type=jax.ShapeDtypeStruct((D2_TOT,), jnp.float32),
    mesh=_MESH,
    compiler_params=_SC_PARAMS,
    scratch_types=(
        [pltpu.VMEM((N_PAD,), jnp.float32)] * 3
        + [pltpu.VMEM((CHUNK,), jnp.int32)] * 4
        + [pltpu.VMEM((CHUNK,), jnp.float32)] * 2
        + [pltpu.SemaphoreType.DMA] * 4
    ),
)(_d2_body)


def _make_scatter_pipelined(n_in, n_acc, chunk):
    per_w = n_in // NW
    steps = per_w // chunk
    assert steps % 4 == 0
    rpt = n_acc // 16

    def body(val_hbm, idx_hbm, zeros_hbm, out_hbm,
             ic0, vb0, ic1, vb1, ic2, vb2, ic3, vb3, acc,
             sf0, sf1, sf2, sf3, ss0, ss1, ss2, ss3):
        c = lax.axis_index("c")
        s = lax.axis_index("s")
        base = (c * 16 + s) * per_w
        SL = [(ic0, vb0, sf0, ss0), (ic1, vb1, sf1, ss1),
              (ic2, vb2, sf2, ss2), (ic3, vb3, sf3, ss3)]

        def off(i):
            return base + i * chunk

        def fetch_descs(sl, i):
            ic, vb, sf, _ = SL[sl]
            return (pltpu.make_async_copy(val_hbm.at[pl.ds(off(i), chunk)], vb, sf),
                    pltpu.make_async_copy(idx_hbm.at[pl.ds(off(i), chunk)], ic, sf))

        def scat_desc(sl):
            ic, vb, _, ss = SL[sl]
            return pltpu.make_async_copy(vb, acc.at[ic], ss)

        def start(ds):
            for d in ds:
                d.start()

        def wait(ds):
            for d in ds:
                d.wait()

        pltpu.sync_copy(zeros_hbm.at[pl.ds(s * rpt, rpt)],
                        acc.at[pl.ds(s * rpt, rpt)])
        plsc.subcore_barrier()
        start(fetch_descs(0, 0))
        start(fetch_descs(1, 1))

        def kbody(k, carry):
            for sl in range(4):
                i = 4 * k + sl
                wait(fetch_descs(sl, i))
                pltpu.async_copy(SL[sl][1], acc.at[SL[sl][0]],
                                 SL[sl][3], add=True)
                nsl = (sl + 2) % 4

                def drain():
                    scat_desc(nsl).wait()

                def prefetch():
                    start(fetch_descs(nsl, i + 2))
                if sl < 2:
                    pl.when(k >= 1)(drain)
                    prefetch()
                else:
                    drain()
                    pl.when(k < steps // 4 - 1)(prefetch)
            return carry

        lax.fori_loop(0, steps // 4, kbody, 0)
        scat_desc((steps - 2) % 4).wait()
        scat_desc((steps - 1) % 4).wait()
        plsc.subcore_barrier()
        pltpu.sync_copy(acc.at[pl.ds(s * rpt, rpt)],
                        out_hbm.at[c, pl.ds(s * rpt, rpt)])

    return functools.partial(
        pl.kernel,
        out_type=jax.ShapeDtypeStruct((2, n_acc, H), jnp.float32),
        mesh=_MESH,
        compiler_params=_SC_PARAMS,
        scratch_types=(
            [pltpu.VMEM((chunk,), jnp.int32), pltpu.VMEM((chunk, H), jnp.float32)] * 4
            + [pltpu.VMEM_SHARED((n_acc, H), jnp.float32)]
            + [pltpu.SemaphoreType.DMA] * 8
        ),
    )(body)


def _make_scatter_simple(n_in, n_acc, chunk):
    per_w = n_in // NW
    steps = per_w // chunk
    rpt = n_acc // 16

    def body(val_hbm, idx_hbm, zeros_hbm, out_hbm, idx_c, vbuf, acc, sem):
        c = lax.axis_index("c")
        s = lax.axis_index("s")
        base = (c * 16 + s) * per_w
        pltpu.sync_copy(zeros_hbm.at[pl.ds(s * rpt, rpt)],
                        acc.at[pl.ds(s * rpt, rpt)])
        plsc.subcore_barrier()

        def step(i, carry):
            off = base + i * chunk
            pltpu.sync_copy(val_hbm.at[pl.ds(off, chunk)], vbuf)
            pltpu.sync_copy(idx_hbm.at[pl.ds(off, chunk)], idx_c)
            pltpu.sync_copy(vbuf, acc.at[idx_c], add=True)
            return carry

        lax.fori_loop(0, steps, step, 0)
        plsc.subcore_barrier()
        pltpu.sync_copy(acc.at[pl.ds(s * rpt, rpt)],
                        out_hbm.at[c, pl.ds(s * rpt, rpt)])

    return functools.partial(
        pl.kernel,
        out_type=jax.ShapeDtypeStruct((2, n_acc, H), jnp.float32),
        mesh=_MESH,
        compiler_params=_SC_PARAMS,
        scratch_types=[
            pltpu.VMEM((chunk,), jnp.int32),
            pltpu.VMEM((chunk, H), jnp.float32),
            pltpu.VMEM_SHARED((n_acc, H), jnp.float32),
            pltpu.SemaphoreType.DMA,
        ],
    )(body)


_sc_scatter_node = _make_scatter_pipelined(E_PAD, N_PAD, 64)
_sc_scatter_pool = _make_scatter_simple(N_PAD, POOL_ACC, 64)


BLK_E = 512
BLK_N = 512


def _vec_spec(grid_fn=None):
    if grid_fn is None:
        grid_fn = lambda i: (0, 0)
    return pl.BlockSpec((1, H), grid_fn)


def _wspec():
    return pl.BlockSpec((H, H), lambda i: (0, 0))


def _emb_kernel(x_ref, ew_ref, eb_ref, h_ref):
    h_ref[...] = _dot(x_ref[...], ew_ref[...]) + eb_ref[...]


def _tc_emb(x_p, ew, eb):
    grid = (N_PAD // BLK_N,)
    row = pl.BlockSpec((BLK_N, H), lambda i: (i, 0))
    return pl.pallas_call(
        _emb_kernel,
        grid=grid,
        in_specs=[row, _wspec(), _vec_spec()],
        out_specs=row,
        out_shape=jax.ShapeDtypeStruct((N_PAD, H), jnp.float32),
    )(x_p, ew, eb)


def _edge_kernel(s_ref, t_ref, d_ref, w1_ref, b1_ref, g1_ref, be1_ref,
                 w2_ref, b2_ref, g2_ref, be2_ref, m_ref):
    d = jnp.sqrt(d_ref[...])
    lane = lax.broadcasted_iota(jnp.int32, (d_ref.shape[0], H), 1)
    dpad = jnp.where(lane == 0, d, 0.0)
    cat = jnp.concatenate([s_ref[...], t_ref[...], dpad], axis=1)
    m1 = _dot(cat, w1_ref[...]) + b1_ref[...]
    m1 = _ln_swish(m1, g1_ref[...], be1_ref[...])
    m2 = _dot(m1, w2_ref[...]) + b2_ref[...]
    m_ref[...] = _ln_swish(m2, g2_ref[...], be2_ref[...])


def _tc_edge(S, T, d, w1pad, b1, g1, be1, w2, b2, g2, be2):
    grid = (E_PAD // BLK_E,)
    row = pl.BlockSpec((BLK_E, H), lambda i: (i, 0))
    dspec = pl.BlockSpec((BLK_E, 1), lambda i: (i, 0))
    w1s = pl.BlockSpec((3 * H, H), lambda i: (0, 0))
    return pl.pallas_call(
        _edge_kernel,
        grid=grid,
        in_specs=[row, row, dspec, w1s, _vec_spec(), _vec_spec(),
                  _vec_spec(), _wspec(), _vec_spec(), _vec_spec(),
                  _vec_spec()],
        out_specs=row,
        out_shape=jax.ShapeDtypeStruct((E_PAD, H), jnp.float32),
    )(S, T, d, w1pad, b1, g1, be1, w2, b2, g2, be2)


def _node_kernel(has_res, *refs):
    i = iter(refs)
    h_ref = next(i)
    a0_ref = next(i)
    a1_ref = next(i)
    h0_ref = next(i) if has_res else None
    wu1 = next(i)[...]
    bu1 = next(i)[...]
    gu1 = next(i)[...]
    beu1 = next(i)[...]
    wu2 = next(i)[...]
    bu2 = next(i)[...]
    gu2 = next(i)[...]
    beu2 = next(i)[...]
    hn_ref = next(i)
    h = h_ref[...]
    agg = a0_ref[0] + a1_ref[0]
    cat = jnp.concatenate([h, agg], axis=1)
    u1 = _ln_swish(_dot(cat, wu1) + bu1, gu1, beu1)
    u2 = _ln_swish(_dot(u1, wu2) + bu2, gu2, beu2)
    if has_res:
        u2 = u2 + h0_ref[...]
    hn_ref[...] = u2


def _tc_node(h, agg2, p, h0):
    has_res = h0 is not None
    grid = (N_PAD // BLK_N,)
    row = pl.BlockSpec((BLK_N, H), lambda i: (i, 0))
    arow = pl.BlockSpec((1, BLK_N, H), lambda i: (0, i, 0))
    brow = pl.BlockSpec((1, BLK_N, H), lambda i: (1, i, 0))
    in_specs = [row, arow, brow]
    args = [h, agg2, agg2]
    if has_res:
        in_specs.append(row)
        args.append(h0)
    for spec, arg in [
        (pl.BlockSpec((2 * H, H), lambda i: (0, 0)), p["upd_W1"]),
        (_vec_spec(), p["upd_b1"].reshape(1, H)),
        (_vec_spec(), p["upd_g1"].reshape(1, H)),
        (_vec_spec(), p["upd_be1"].reshape(1, H)),
        (_wspec(), p["upd_W2"]),
        (_vec_spec(), p["upd_b2"].reshape(1, H)),
        (_vec_spec(), p["upd_g2"].reshape(1, H)),
        (_vec_spec(), p["upd_be2"].reshape(1, H)),
    ]:
        in_specs.append(spec)
        args.append(arg)
    return pl.pallas_call(
        functools.partial(_node_kernel, has_res),
        grid=grid,
        in_specs=in_specs,
        out_specs=row,
        out_shape=jax.ShapeDtypeStruct((N_PAD, H), jnp.float32),
    )(*args)


def _pred_kernel(p2_ref, w1_ref, b1_ref, w2_ref, b2_ref, o_ref):
    pooled = p2_ref[0, :NGRAPH] + p2_ref[1, :NGRAPH]
    hid = jnp.maximum(_dot(pooled, w1_ref[...]) + b1_ref[...], 0.0)
    o_ref[...] = _dot(hid, w2_ref[...]) + b2_ref[...]


def _tc_pred(pooled2, w1, b1, w2, b2):
    return pl.pallas_call(
        _pred_kernel,
        in_specs=[
            pl.BlockSpec((2, POOL_ACC, H), lambda: (0, 0, 0)),
            _wspec_g(), pl.BlockSpec((1, H), lambda: (0, 0)),
            pl.BlockSpec((H, 1), lambda: (0, 0)),
            pl.BlockSpec((1, 1), lambda: (0, 0)),
        ],
        out_specs=pl.BlockSpec((NGRAPH, 1), lambda: (0, 0)),
        out_shape=jax.ShapeDtypeStruct((NGRAPH, 1), jnp.float32),
    )(pooled2, w1, b1.reshape(1, H), w2, b2.reshape(1, 1))


def _wspec_g():
    return pl.BlockSpec((H, H), lambda: (0, 0))


def kernel(x, pos, edge_index, node_subnode_index, subgraph_edge_index,
           subnode_node_index, batch, params):
    f32 = jnp.float32
    x_p = jnp.pad(x.astype(f32), ((0, N_PAD - N), (0, 0)))
    pos16 = jnp.pad(pos.astype(f32), ((0, N_PAD - N), (0, 13)))

    edge_sets = [edge_index, node_subnode_index, subgraph_edge_index,
                 subnode_node_index]
    dsts, srcs = [], []
    for es in edge_sets:
        es = es.astype(jnp.int32)
        pad = jnp.full((E_PAD - E,), N, jnp.int32)
        srcs.append(jnp.concatenate([es[0], pad]))
        dsts.append(jnp.concatenate([es[1], pad]))

    batch_pad = jnp.concatenate([
        batch.astype(jnp.int32),
        jnp.full((N_PAD - N,), NGRAPH, jnp.int32)])

    zeros_node = jnp.zeros((N_PAD, H), f32)
    zeros_pool = jnp.zeros((POOL_ACC, H), f32)

    pos_p = pos16[:, :3]
    dst_all = jnp.concatenate(dsts)
    src_all = jnp.concatenate(srcs)
    d2_flat = _sc_d2(pos_p[:, 0], pos_p[:, 1], pos_p[:, 2],
                     dst_all, src_all)
    d2_all = d2_flat.reshape(4, E_PAD, 1)
    d_list = [d2_all[t] for t in range(4)]

    prm = params
    lay = prm["layers"]

    h = _tc_emb(x_p, prm["emb_W"], prm["emb_b"].reshape(1, H))

    for l in range(DEPTH):
        h0 = h
        for t in range(4):
            p = lay[l][t]
            S, T = _sc_gather_pair(h, h, dsts[t], srcs[t])
            m = _tc_edge(
                S, T, d_list[t],
                jnp.pad(p["msg_W1"], ((0, 3 * H - (2 * H + 1)), (0, 0))),
                p["msg_b1"].reshape(1, H),
                p["msg_g1"].reshape(1, H), p["msg_be1"].reshape(1, H),
                p["msg_W2"], p["msg_b2"].reshape(1, H),
                p["msg_g2"].reshape(1, H), p["msg_be2"].reshape(1, H))
            agg2 = _sc_scatter_node(m, dsts[t], zeros_node)
            h = _tc_node(h, agg2, p, h0 if t == 3 else None)

    pooled2 = _sc_scatter_pool(h, batch_pad, zeros_pool)
    return _tc_pred(pooled2, prm["pred_W1"], prm["pred_b1"],
                    prm["pred_W2"], prm["pred_b2"])
